# baseline (speedup 1.0000x reference)
import functools

import jax
import jax.numpy as jnp
import numpy as np
from jax import lax
from jax.experimental import pallas as pl
from jax.experimental.pallas import tpu as pltpu
from jax.experimental.pallas import tpu_sc as plsc

NCELL = 49
D = 30
BATCH = 4096
NW = 32
LANES = BATCH // NW
CELLS_PER_CHUNK = 7
N_CHUNKS = NCELL // CELLS_PER_CHUNK
GROUPS = LANES // 16
PCH = 16
TCH = 8
INV_GS = np.float32(1.0 / 7.0)
BS = np.float32(4096.0)


def _sqrt16(x):
    i = lax.bitcast_convert_type(x, jnp.int32)
    y = lax.bitcast_convert_type(jnp.int32(0x5F3759DF) - (i >> 1), jnp.float32)
    half_x = 0.5 * x
    for _ in range(2):
        y = y * (1.5 - half_x * y * y)
    return x * y


def _box_kernel(p_hbm, t_hbm, out_hbm, pbuf0, pbuf1, tbuf0, tbuf1, outv, psems, tsems):
    wid = lax.axis_index("s") * 2 + lax.axis_index("c")
    b0 = wid * LANES
    pbufs = (pbuf0, pbuf1)
    tbufs = (tbuf0, tbuf1)

    def start_copy(chunk, slot):
        c0 = chunk * CELLS_PER_CHUNK
        pc = pltpu.async_copy(
            p_hbm.at[pl.ds(c0, CELLS_PER_CHUNK), pl.ds(0, PCH), pl.ds(b0, LANES)],
            pbufs[slot],
            psems[slot],
        )
        tc = pltpu.async_copy(
            t_hbm.at[pl.ds(c0, CELLS_PER_CHUNK), pl.ds(0, TCH), pl.ds(b0, LANES)],
            tbufs[slot],
            tsems[slot],
        )
        return pc, tc

    def cell_group(pref, tref, cc, g, accs):
        acc_b, acc_c = accs
        sl = pl.ds(g * 16, 16)

        def gp(c):
            return pref[cc, c, sl]

        def gt(c):
            return tref[cc, c, sl]

        p0, p1, p2, p3, p4 = gp(0), gp(1), gp(2), gp(3), gp(4)
        p5, p6, p7, p8, p9 = gp(5), gp(6), gp(7), gp(8), gp(9)
        t0, t1, t2, t3, t4 = gt(0), gt(1), gt(2), gt(3), gt(4)
        obj = t4

        tx = t0 * INV_GS
        ty = t1 * INV_GS
        thw = 0.5 * t2
        thh = 0.5 * t3
        tx1, tx2 = tx - thw, tx + thw
        ty1, ty2 = ty - thh, ty + thh
        area_t = (tx2 - tx1) * (ty2 - ty1)

        def iou_box(px, py, pw, ph):
            x = px * INV_GS
            y = py * INV_GS
            hw = 0.5 * pw
            hh = 0.5 * ph
            x1, x2 = x - hw, x + hw
            y1, y2 = y - hh, y + hh
            wix = jnp.maximum(jnp.minimum(x2, tx2) - jnp.maximum(x1, tx1), 0.0)
            wiy = jnp.maximum(jnp.minimum(y2, ty2) - jnp.maximum(y1, ty1), 0.0)
            inter = wix * wiy
            area_p = (x2 - x1) * (y2 - y1)
            return inter / (area_p + area_t - inter)

        iou0 = iou_box(p0, p1, p2, p3)
        iou1 = iou_box(p5, p6, p7, p8)
        r0 = iou0 >= iou1

        def sel(a, b):
            return jnp.where(r0, a, b)

        dx = sel(p0, p5) - t0
        dy = sel(p1, p6) - t1
        dw = _sqrt16(sel(p2, p7)) - _sqrt16(t2)
        dh = _sqrt16(sel(p3, p8)) - _sqrt16(t3)
        acc_b = acc_b + obj * (dx * dx + dy * dy + dw * dw + dh * dh)

        dob = sel(p4, p9) - jnp.maximum(iou0, iou1)
        acc_c = acc_c + obj * (dob * dob)
        return acc_b, acc_c

    zeros = jnp.zeros((16,), jnp.float32)
    accs = (zeros, zeros)
    copies = start_copy(0, 0)
    for chunk in range(N_CHUNKS):
        slot = chunk % 2
        pc, tc = copies
        pc.wait()
        tc.wait()
        if chunk + 1 < N_CHUNKS:
            copies = start_copy(chunk + 1, 1 - slot)

        @plsc.parallel_loop(0, CELLS_PER_CHUNK * GROUPS, unroll=2, carry=accs)
        def accs(i, a, _slot=slot):
            cc = i // GROUPS
            g = i - cc * GROUPS
            return cell_group(pbufs[_slot], tbufs[_slot], cc, g, a)

    acc_b, acc_c = accs
    outv[pl.ds(0, 16)] = acc_b * (np.float32(0.5) / BS)
    outv[pl.ds(16, 16)] = acc_c * (np.float32(1.0) / BS)
    pltpu.sync_copy(outv, out_hbm.at[wid])


@jax.jit
def _run_sc(p_cells, t_cells):
    mesh = plsc.VectorSubcoreMesh(core_axis_name="c", subcore_axis_name="s")
    kern = functools.partial(
        pl.kernel,
        out_type=jax.ShapeDtypeStruct((NW, 32), jnp.float32),
        mesh=mesh,
        scratch_types=[
            pltpu.VMEM((CELLS_PER_CHUNK, PCH, LANES), jnp.float32),
            pltpu.VMEM((CELLS_PER_CHUNK, PCH, LANES), jnp.float32),
            pltpu.VMEM((CELLS_PER_CHUNK, TCH, LANES), jnp.float32),
            pltpu.VMEM((CELLS_PER_CHUNK, TCH, LANES), jnp.float32),
            pltpu.VMEM((32,), jnp.float32),
            (pltpu.SemaphoreType.DMA, pltpu.SemaphoreType.DMA),
            (pltpu.SemaphoreType.DMA, pltpu.SemaphoreType.DMA),
        ],
        compiler_params=pltpu.CompilerParams(
            use_tc_tiling_on_sc=True, skip_device_barrier=True
        ),
    )(_box_kernel)
    return kern(p_cells, t_cells)


def _dense_kernel(p_ref, t_ref, out_ref):
    p = p_ref[0]
    t = t_ref[0]
    obj = t[4:5, :]
    dcls = p[10:30, :] - t[10:30, :]
    cls_cell = jnp.sum((dcls * dcls) * obj)
    pc = p[4:5, :]
    pc2 = p[9:10, :]
    noobj_cell = jnp.sum((1.0 - obj) * (pc * pc + pc2 * pc2))
    out_ref[0, 0, 0] = cls_cell
    out_ref[0, 0, 1] = noobj_cell


@jax.jit
def _run_tc(p_cells, t_cells):
    return pl.pallas_call(
        _dense_kernel,
        grid=(NCELL,),
        in_specs=[
            pl.BlockSpec((1, D, BATCH), lambda i: (i, 0, 0)),
            pl.BlockSpec((1, D, BATCH), lambda i: (i, 0, 0)),
        ],
        out_specs=pl.BlockSpec(
            (1, 1, 2), lambda i: (i, 0, 0), memory_space=pltpu.SMEM
        ),
        out_shape=jax.ShapeDtypeStruct((NCELL, 1, 2), jnp.float32),
    )(p_cells, t_cells)


def kernel(inputs, targets):
    p_cells = jnp.transpose(inputs, (1, 2, 3, 0)).reshape(NCELL, D, BATCH)
    t_cells = jnp.transpose(targets, (1, 2, 3, 0)).reshape(NCELL, D, BATCH)
    sc_part = _run_sc(p_cells, t_cells)
    tc_part = _run_tc(p_cells, t_cells)
    sc_sums = sc_part.reshape(NW, 2, 16).sum(axis=(0, 2))
    tc_sums = tc_part.sum(axis=(0, 1)) * (np.float32(1.0) / BS)
    loss_boxes = sc_sums[0]
    loss_conf = sc_sums[1] + np.float32(0.5) * tc_sums[1]
    loss_class = tc_sums[0]
    return jnp.stack([loss_boxes, loss_conf, loss_class])


# speedup vs baseline: 1.0993x; 1.0993x over previous
import functools

import jax
import jax.numpy as jnp
import numpy as np
from jax import lax
from jax.experimental import pallas as pl
from jax.experimental.pallas import tpu as pltpu
from jax.experimental.pallas import tpu_sc as plsc

NCELL = 49
D = 30
BATCH = 4096
NW = 32
LANES = BATCH // NW
CELLS_PER_CHUNK = 7
N_CHUNKS = NCELL // CELLS_PER_CHUNK
GROUPS = LANES // 16
SCH = 16
INV_GS = np.float32(1.0 / 7.0)
BS = np.float32(4096.0)


def _sqrt16(x):
    i = lax.bitcast_convert_type(x, jnp.int32)
    y = lax.bitcast_convert_type(jnp.int32(0x5F3759DF) - (i >> 1), jnp.float32)
    half_x = 0.5 * x
    for _ in range(2):
        y = y * (1.5 - half_x * y * y)
    return x * y


def _box_kernel(p_hbm, t_hbm, out_hbm, pbuf0, pbuf1, tbuf0, tbuf1, outv, psems, tsems):
    wid = lax.axis_index("s") * 2 + lax.axis_index("c")
    b0 = wid * LANES
    pbufs = (pbuf0, pbuf1)
    tbufs = (tbuf0, tbuf1)

    def start_copy(chunk, slot):
        c0 = chunk * CELLS_PER_CHUNK
        pc = pltpu.async_copy(
            p_hbm.at[pl.ds(c0, CELLS_PER_CHUNK), pl.ds(0, SCH), pl.ds(b0, LANES)],
            pbufs[slot],
            psems[slot],
        )
        tc = pltpu.async_copy(
            t_hbm.at[pl.ds(c0, CELLS_PER_CHUNK), pl.ds(0, SCH), pl.ds(b0, LANES)],
            tbufs[slot],
            tsems[slot],
        )
        return pc, tc

    def cell_group(pref, tref, cc, g, accs):
        acc_b, acc_c, acc_k = accs
        sl = pl.ds(g * 16, 16)

        def gp(c):
            return pref[cc, c, sl]

        def gt(c):
            return tref[cc, c, sl]

        p0, p1, p2, p3, p4 = gp(0), gp(1), gp(2), gp(3), gp(4)
        p5, p6, p7, p8, p9 = gp(5), gp(6), gp(7), gp(8), gp(9)
        t0, t1, t2, t3, t4 = gt(0), gt(1), gt(2), gt(3), gt(4)
        obj = t4
        noobj = 1.0 - t4

        tx = t0 * INV_GS
        ty = t1 * INV_GS
        thw = 0.5 * t2
        thh = 0.5 * t3
        tx1, tx2 = tx - thw, tx + thw
        ty1, ty2 = ty - thh, ty + thh
        area_t = (tx2 - tx1) * (ty2 - ty1)

        def iou_box(px, py, pw, ph):
            x = px * INV_GS
            y = py * INV_GS
            hw = 0.5 * pw
            hh = 0.5 * ph
            x1, x2 = x - hw, x + hw
            y1, y2 = y - hh, y + hh
            wix = jnp.maximum(jnp.minimum(x2, tx2) - jnp.maximum(x1, tx1), 0.0)
            wiy = jnp.maximum(jnp.minimum(y2, ty2) - jnp.maximum(y1, ty1), 0.0)
            inter = wix * wiy
            area_p = (x2 - x1) * (y2 - y1)
            return inter / (area_p + area_t - inter)

        iou0 = iou_box(p0, p1, p2, p3)
        iou1 = iou_box(p5, p6, p7, p8)
        r0 = iou0 >= iou1

        def sel(a, b):
            return jnp.where(r0, a, b)

        dx = sel(p0, p5) - t0
        dy = sel(p1, p6) - t1
        dw = _sqrt16(sel(p2, p7)) - _sqrt16(t2)
        dh = _sqrt16(sel(p3, p8)) - _sqrt16(t3)
        acc_b = acc_b + obj * (dx * dx + dy * dy + dw * dw + dh * dh)

        dob = sel(p4, p9) - jnp.maximum(iou0, iou1)
        acc_c = acc_c + obj * (dob * dob) + (0.5 * noobj) * (p4 * p4 + p9 * p9)

        s = jnp.zeros((16,), jnp.float32)
        for c in range(10, SCH):
            d = gp(c) - gt(c)
            s = s + d * d
        acc_k = acc_k + obj * s
        return acc_b, acc_c, acc_k

    zeros = jnp.zeros((16,), jnp.float32)
    accs = (zeros, zeros, zeros)
    copies = start_copy(0, 0)
    for chunk in range(N_CHUNKS):
        slot = chunk % 2
        pc, tc = copies
        pc.wait()
        tc.wait()
        if chunk + 1 < N_CHUNKS:
            copies = start_copy(chunk + 1, 1 - slot)

        @plsc.parallel_loop(0, CELLS_PER_CHUNK * GROUPS, unroll=2, carry=accs)
        def accs(i, a, _slot=slot):
            cc = i // GROUPS
            g = i - cc * GROUPS
            return cell_group(pbufs[_slot], tbufs[_slot], cc, g, a)

    acc_b, acc_c, acc_k = accs
    outv[pl.ds(0, 16)] = acc_b * (np.float32(0.5) / BS)
    outv[pl.ds(16, 16)] = acc_c * (np.float32(1.0) / BS)
    outv[pl.ds(32, 16)] = acc_k * (np.float32(1.0) / BS)
    pltpu.sync_copy(outv, out_hbm.at[wid])


@jax.jit
def _run_sc(p_cells, t_cells):
    mesh = plsc.VectorSubcoreMesh(core_axis_name="c", subcore_axis_name="s")
    kern = functools.partial(
        pl.kernel,
        out_type=jax.ShapeDtypeStruct((NW, 48), jnp.float32),
        mesh=mesh,
        scratch_types=[
            pltpu.VMEM((CELLS_PER_CHUNK, SCH, LANES), jnp.float32),
            pltpu.VMEM((CELLS_PER_CHUNK, SCH, LANES), jnp.float32),
            pltpu.VMEM((CELLS_PER_CHUNK, SCH, LANES), jnp.float32),
            pltpu.VMEM((CELLS_PER_CHUNK, SCH, LANES), jnp.float32),
            pltpu.VMEM((48,), jnp.float32),
            (pltpu.SemaphoreType.DMA, pltpu.SemaphoreType.DMA),
            (pltpu.SemaphoreType.DMA, pltpu.SemaphoreType.DMA),
        ],
        compiler_params=pltpu.CompilerParams(
            use_tc_tiling_on_sc=True, skip_device_barrier=True
        ),
    )(_box_kernel)
    return kern(p_cells, t_cells)


def _dense_kernel(p2_ref, p3_ref, t2_ref, t3_ref, t0_ref, out_ref):
    obj = t0_ref[0, 4:5, :]
    d2 = p2_ref[0] - t2_ref[0]
    s2 = jnp.sum((d2 * d2) * obj)
    d3 = p3_ref[0] - t3_ref[0]
    row = lax.broadcasted_iota(jnp.int32, d3.shape, 0)
    sq3 = jnp.where(row < 6, d3 * d3, 0.0)
    s3 = jnp.sum(sq3 * obj)
    out_ref[0, 0, 0] = s2 + s3


@jax.jit
def _run_tc(p_cells, t_cells):
    return pl.pallas_call(
        _dense_kernel,
        grid=(NCELL,),
        in_specs=[
            pl.BlockSpec((1, 8, BATCH), lambda i: (i, 2, 0)),
            pl.BlockSpec((1, 8, BATCH), lambda i: (i, 3, 0)),
            pl.BlockSpec((1, 8, BATCH), lambda i: (i, 2, 0)),
            pl.BlockSpec((1, 8, BATCH), lambda i: (i, 3, 0)),
            pl.BlockSpec((1, 8, BATCH), lambda i: (i, 0, 0)),
        ],
        out_specs=pl.BlockSpec(
            (1, 1, 1), lambda i: (i, 0, 0), memory_space=pltpu.SMEM
        ),
        out_shape=jax.ShapeDtypeStruct((NCELL, 1, 1), jnp.float32),
    )(p_cells, p_cells, t_cells, t_cells, t_cells)


def kernel(inputs, targets):
    p_cells = jnp.transpose(inputs, (1, 2, 3, 0)).reshape(NCELL, D, BATCH)
    t_cells = jnp.transpose(targets, (1, 2, 3, 0)).reshape(NCELL, D, BATCH)
    sc_part = _run_sc(p_cells, t_cells)
    tc_part = _run_tc(p_cells, t_cells)
    sc_sums = sc_part.reshape(NW, 3, 16).sum(axis=(0, 2))
    tc_cls = tc_part.sum() * (np.float32(1.0) / BS)
    loss_boxes = sc_sums[0]
    loss_conf = sc_sums[1]
    loss_class = sc_sums[2] + tc_cls
    return jnp.stack([loss_boxes, loss_conf, loss_class])


# speedup vs baseline: 1.1390x; 1.0362x over previous
import functools

import jax
import jax.numpy as jnp
import numpy as np
from jax import lax
from jax.experimental import pallas as pl
from jax.experimental.pallas import tpu as pltpu
from jax.experimental.pallas import tpu_sc as plsc

NCELL = 49
D = 30
BATCH = 4096
NW = 32
LANES = BATCH // NW
CELLS_PER_CHUNK = 7
N_CHUNKS = NCELL // CELLS_PER_CHUNK
GROUPS = LANES // 16
SCH = 24
INV_GS = np.float32(1.0 / 7.0)
BS = np.float32(4096.0)


def _sqrt16(x):
    i = lax.bitcast_convert_type(x, jnp.int32)
    y = lax.bitcast_convert_type(jnp.int32(0x5F3759DF) - (i >> 1), jnp.float32)
    half_x = 0.5 * x
    for _ in range(2):
        y = y * (1.5 - half_x * y * y)
    return x * y


def _box_kernel(p_hbm, t_hbm, out_hbm, pbuf0, pbuf1, tbuf0, tbuf1, outv, psems, tsems):
    wid = lax.axis_index("s") * 2 + lax.axis_index("c")
    b0 = wid * LANES
    pbufs = (pbuf0, pbuf1)
    tbufs = (tbuf0, tbuf1)

    def start_copy(chunk, slot):
        c0 = chunk * CELLS_PER_CHUNK
        pc = pltpu.async_copy(
            p_hbm.at[pl.ds(c0, CELLS_PER_CHUNK), pl.ds(0, SCH), pl.ds(b0, LANES)],
            pbufs[slot],
            psems[slot],
        )
        tc = pltpu.async_copy(
            t_hbm.at[pl.ds(c0, CELLS_PER_CHUNK), pl.ds(0, SCH), pl.ds(b0, LANES)],
            tbufs[slot],
            tsems[slot],
        )
        return pc, tc

    def cell_group(pref, tref, cc, g, accs):
        acc_b, acc_c, acc_k = accs
        sl = pl.ds(g * 16, 16)

        def gp(c):
            return pref[cc, c, sl]

        def gt(c):
            return tref[cc, c, sl]

        p0, p1, p2, p3, p4 = gp(0), gp(1), gp(2), gp(3), gp(4)
        p5, p6, p7, p8, p9 = gp(5), gp(6), gp(7), gp(8), gp(9)
        t0, t1, t2, t3, t4 = gt(0), gt(1), gt(2), gt(3), gt(4)
        obj = t4
        noobj = 1.0 - t4

        tx = t0 * INV_GS
        ty = t1 * INV_GS
        thw = 0.5 * t2
        thh = 0.5 * t3
        tx1, tx2 = tx - thw, tx + thw
        ty1, ty2 = ty - thh, ty + thh
        area_t = (tx2 - tx1) * (ty2 - ty1)

        def iou_box(px, py, pw, ph):
            x = px * INV_GS
            y = py * INV_GS
            hw = 0.5 * pw
            hh = 0.5 * ph
            x1, x2 = x - hw, x + hw
            y1, y2 = y - hh, y + hh
            wix = jnp.maximum(jnp.minimum(x2, tx2) - jnp.maximum(x1, tx1), 0.0)
            wiy = jnp.maximum(jnp.minimum(y2, ty2) - jnp.maximum(y1, ty1), 0.0)
            inter = wix * wiy
            area_p = (x2 - x1) * (y2 - y1)
            return inter / (area_p + area_t - inter)

        iou0 = iou_box(p0, p1, p2, p3)
        iou1 = iou_box(p5, p6, p7, p8)
        r0 = iou0 >= iou1

        def sel(a, b):
            return jnp.where(r0, a, b)

        dx = sel(p0, p5) - t0
        dy = sel(p1, p6) - t1
        dw = _sqrt16(sel(p2, p7)) - _sqrt16(t2)
        dh = _sqrt16(sel(p3, p8)) - _sqrt16(t3)
        acc_b = acc_b + obj * (dx * dx + dy * dy + dw * dw + dh * dh)

        dob = sel(p4, p9) - jnp.maximum(iou0, iou1)
        acc_c = acc_c + obj * (dob * dob) + (0.5 * noobj) * (p4 * p4 + p9 * p9)

        s = jnp.zeros((16,), jnp.float32)
        for c in range(10, SCH):
            d = gp(c) - gt(c)
            s = s + d * d
        acc_k = acc_k + obj * s
        return acc_b, acc_c, acc_k

    zeros = jnp.zeros((16,), jnp.float32)
    accs = (zeros, zeros, zeros)
    copies = start_copy(0, 0)
    for chunk in range(N_CHUNKS):
        slot = chunk % 2
        pc, tc = copies
        pc.wait()
        tc.wait()
        if chunk + 1 < N_CHUNKS:
            copies = start_copy(chunk + 1, 1 - slot)

        @plsc.parallel_loop(0, CELLS_PER_CHUNK * GROUPS, unroll=2, carry=accs)
        def accs(i, a, _slot=slot):
            cc = i // GROUPS
            g = i - cc * GROUPS
            return cell_group(pbufs[_slot], tbufs[_slot], cc, g, a)

    acc_b, acc_c, acc_k = accs
    outv[pl.ds(0, 16)] = acc_b * (np.float32(0.5) / BS)
    outv[pl.ds(16, 16)] = acc_c * (np.float32(1.0) / BS)
    outv[pl.ds(32, 16)] = acc_k * (np.float32(1.0) / BS)
    pltpu.sync_copy(outv, out_hbm.at[wid])


@jax.jit
def _run_sc(p_cells, t_cells):
    mesh = plsc.VectorSubcoreMesh(core_axis_name="c", subcore_axis_name="s")
    kern = functools.partial(
        pl.kernel,
        out_type=jax.ShapeDtypeStruct((NW, 48), jnp.float32),
        mesh=mesh,
        scratch_types=[
            pltpu.VMEM((CELLS_PER_CHUNK, SCH, LANES), jnp.float32),
            pltpu.VMEM((CELLS_PER_CHUNK, SCH, LANES), jnp.float32),
            pltpu.VMEM((CELLS_PER_CHUNK, SCH, LANES), jnp.float32),
            pltpu.VMEM((CELLS_PER_CHUNK, SCH, LANES), jnp.float32),
            pltpu.VMEM((48,), jnp.float32),
            (pltpu.SemaphoreType.DMA, pltpu.SemaphoreType.DMA),
            (pltpu.SemaphoreType.DMA, pltpu.SemaphoreType.DMA),
        ],
        compiler_params=pltpu.CompilerParams(
            use_tc_tiling_on_sc=True, skip_device_barrier=True
        ),
    )(_box_kernel)
    return kern(p_cells, t_cells)


def _dense_kernel(p3_ref, t3_ref, t0_ref, out_ref):
    obj = t0_ref[0, 4:5, :]
    d3 = p3_ref[0] - t3_ref[0]
    row = lax.broadcasted_iota(jnp.int32, d3.shape, 0)
    sq3 = jnp.where(row < 6, d3 * d3, 0.0)
    out_ref[0, 0, 0] = jnp.sum(sq3 * obj)


@jax.jit
def _run_tc(p_cells, t_cells):
    return pl.pallas_call(
        _dense_kernel,
        grid=(NCELL,),
        in_specs=[
            pl.BlockSpec((1, 8, BATCH), lambda i: (i, 3, 0)),
            pl.BlockSpec((1, 8, BATCH), lambda i: (i, 3, 0)),
            pl.BlockSpec((1, 8, BATCH), lambda i: (i, 0, 0)),
        ],
        out_specs=pl.BlockSpec(
            (1, 1, 1), lambda i: (i, 0, 0), memory_space=pltpu.SMEM
        ),
        out_shape=jax.ShapeDtypeStruct((NCELL, 1, 1), jnp.float32),
    )(p_cells, t_cells, t_cells)


def kernel(inputs, targets):
    p_cells = jnp.transpose(inputs, (1, 2, 3, 0)).reshape(NCELL, D, BATCH)
    t_cells = jnp.transpose(targets, (1, 2, 3, 0)).reshape(NCELL, D, BATCH)
    sc_part = _run_sc(p_cells, t_cells)
    tc_part = _run_tc(p_cells, t_cells)
    sc_sums = sc_part.reshape(NW, 3, 16).sum(axis=(0, 2))
    tc_cls = tc_part.sum() * (np.float32(1.0) / BS)
    loss_boxes = sc_sums[0]
    loss_conf = sc_sums[1]
    loss_class = sc_sums[2] + tc_cls
    return jnp.stack([loss_boxes, loss_conf, loss_class])


# speedup vs baseline: 1.2886x; 1.1313x over previous
import functools

import jax
import jax.numpy as jnp
import numpy as np
from jax import lax
from jax.experimental import pallas as pl
from jax.experimental.pallas import tpu as pltpu
from jax.experimental.pallas import tpu_sc as plsc

NCELL = 49
D = 30
BATCH = 4096
NW = 32
LANES = BATCH // NW
CELLS_PER_CHUNK = 7
N_CHUNKS = NCELL // CELLS_PER_CHUNK
GROUPS = LANES // 16
INV_GS = np.float32(1.0 / 7.0)
BS = np.float32(4096.0)


def _sqrt16(x):
    i = lax.bitcast_convert_type(x, jnp.int32)
    y = lax.bitcast_convert_type(jnp.int32(0x5F3759DF) - (i >> 1), jnp.float32)
    half_x = 0.5 * x
    for _ in range(2):
        y = y * (1.5 - half_x * y * y)
    return x * y


def _loss_kernel(
    p_hbm, t_hbm, out_hbm,
    pbuf0, pbuf1, tbuf0, tbuf1, kbuf0, kbuf1, outv,
    psems, tsems, ksems,
):
    wid = lax.axis_index("s") * 2 + lax.axis_index("c")
    b0 = wid * LANES
    pbufs = (pbuf0, pbuf1)
    tbufs = (tbuf0, tbuf1)
    kbufs = (kbuf0, kbuf1)

    def start_copy(chunk, slot):
        c0 = chunk * CELLS_PER_CHUNK
        pc = pltpu.async_copy(
            p_hbm.at[pl.ds(c0, CELLS_PER_CHUNK), :, pl.ds(b0, LANES)],
            pbufs[slot],
            psems[slot],
        )
        tc = pltpu.async_copy(
            t_hbm.at[pl.ds(c0, CELLS_PER_CHUNK), pl.ds(0, 5), pl.ds(b0, LANES)],
            tbufs[slot],
            tsems[slot],
        )
        kc = pltpu.async_copy(
            t_hbm.at[pl.ds(c0, CELLS_PER_CHUNK), pl.ds(8, 22), pl.ds(b0, LANES)],
            kbufs[slot],
            ksems[slot],
        )
        return pc, tc, kc

    def cell_group(pref, tref, kref, cc, g, accs):
        acc_b, acc_c, acc_k = accs
        sl = pl.ds(g * 16, 16)

        def gp(c):
            return pref[cc, c, sl]

        def gt(c):
            return tref[cc, c, sl]

        def gk(c):
            return kref[cc, c - 8, sl]

        p0, p1, p2, p3, p4 = gp(0), gp(1), gp(2), gp(3), gp(4)
        p5, p6, p7, p8, p9 = gp(5), gp(6), gp(7), gp(8), gp(9)
        t0, t1, t2, t3, t4 = gt(0), gt(1), gt(2), gt(3), gt(4)
        obj = t4
        noobj = 1.0 - t4

        tx = t0 * INV_GS
        ty = t1 * INV_GS
        thw = 0.5 * t2
        thh = 0.5 * t3
        tx1, tx2 = tx - thw, tx + thw
        ty1, ty2 = ty - thh, ty + thh
        area_t = (tx2 - tx1) * (ty2 - ty1)

        def iou_box(px, py, pw, ph):
            x = px * INV_GS
            y = py * INV_GS
            hw = 0.5 * pw
            hh = 0.5 * ph
            x1, x2 = x - hw, x + hw
            y1, y2 = y - hh, y + hh
            wix = jnp.maximum(jnp.minimum(x2, tx2) - jnp.maximum(x1, tx1), 0.0)
            wiy = jnp.maximum(jnp.minimum(y2, ty2) - jnp.maximum(y1, ty1), 0.0)
            inter = wix * wiy
            area_p = (x2 - x1) * (y2 - y1)
            return inter / (area_p + area_t - inter)

        iou0 = iou_box(p0, p1, p2, p3)
        iou1 = iou_box(p5, p6, p7, p8)
        r0 = iou0 >= iou1

        def sel(a, b):
            return jnp.where(r0, a, b)

        dx = sel(p0, p5) - t0
        dy = sel(p1, p6) - t1
        dw = _sqrt16(sel(p2, p7)) - _sqrt16(t2)
        dh = _sqrt16(sel(p3, p8)) - _sqrt16(t3)
        acc_b = acc_b + obj * (dx * dx + dy * dy + dw * dw + dh * dh)

        dob = sel(p4, p9) - jnp.maximum(iou0, iou1)
        acc_c = acc_c + obj * (dob * dob) + (0.5 * noobj) * (p4 * p4 + p9 * p9)

        s = jnp.zeros((16,), jnp.float32)
        for c in range(10, 30):
            d = gp(c) - gk(c)
            s = s + d * d
        acc_k = acc_k + obj * s
        return acc_b, acc_c, acc_k

    zeros = jnp.zeros((16,), jnp.float32)
    accs = (zeros, zeros, zeros)
    copies = start_copy(0, 0)
    for chunk in range(N_CHUNKS):
        slot = chunk % 2
        pc, tc, kc = copies
        pc.wait()
        tc.wait()
        kc.wait()
        if chunk + 1 < N_CHUNKS:
            copies = start_copy(chunk + 1, 1 - slot)

        @plsc.parallel_loop(0, CELLS_PER_CHUNK * GROUPS, unroll=2, carry=accs)
        def accs(i, a, _slot=slot):
            cc = i // GROUPS
            g = i - cc * GROUPS
            return cell_group(pbufs[_slot], tbufs[_slot], kbufs[_slot], cc, g, a)

    acc_b, acc_c, acc_k = accs
    outv[pl.ds(0, 16)] = acc_b * (np.float32(0.5) / BS)
    outv[pl.ds(16, 16)] = acc_c * (np.float32(1.0) / BS)
    outv[pl.ds(32, 16)] = acc_k * (np.float32(1.0) / BS)
    pltpu.sync_copy(outv, out_hbm.at[wid])


@jax.jit
def _run(p_cells, t_cells):
    mesh = plsc.VectorSubcoreMesh(core_axis_name="c", subcore_axis_name="s")
    kern = functools.partial(
        pl.kernel,
        out_type=jax.ShapeDtypeStruct((NW, 48), jnp.float32),
        mesh=mesh,
        scratch_types=[
            pltpu.VMEM((CELLS_PER_CHUNK, D, LANES), jnp.float32),
            pltpu.VMEM((CELLS_PER_CHUNK, D, LANES), jnp.float32),
            pltpu.VMEM((CELLS_PER_CHUNK, 5, LANES), jnp.float32),
            pltpu.VMEM((CELLS_PER_CHUNK, 5, LANES), jnp.float32),
            pltpu.VMEM((CELLS_PER_CHUNK, 22, LANES), jnp.float32),
            pltpu.VMEM((CELLS_PER_CHUNK, 22, LANES), jnp.float32),
            pltpu.VMEM((48,), jnp.float32),
            (pltpu.SemaphoreType.DMA, pltpu.SemaphoreType.DMA),
            (pltpu.SemaphoreType.DMA, pltpu.SemaphoreType.DMA),
            (pltpu.SemaphoreType.DMA, pltpu.SemaphoreType.DMA),
        ],
        compiler_params=pltpu.CompilerParams(
            use_tc_tiling_on_sc=True, skip_device_barrier=True
        ),
    )(_loss_kernel)
    return kern(p_cells, t_cells)


def kernel(inputs, targets):
    p_cells = jnp.transpose(inputs, (1, 2, 3, 0)).reshape(NCELL, D, BATCH)
    t_cells = jnp.transpose(targets, (1, 2, 3, 0)).reshape(NCELL, D, BATCH)
    partials = _run(p_cells, t_cells)
    return partials.reshape(NW, 3, 16).sum(axis=(0, 2))
